# baseline (speedup 1.0000x reference)
import functools

import jax
import jax.numpy as jnp
from jax import lax
from jax.experimental import pallas as pl
from jax.experimental.pallas import tpu as pltpu
from jax.experimental.pallas import tpu_sc as plsc

B = 1024
N1, N2 = 25, 10
DIN = 128
H0, H1 = 256, 128
NODES = 1 + N1 + N1 * N2
BB = 64
PAD = 32

DB = 64
NW = 32
IPW = B // NW
VPR = DIN // 16


def _sc_prep_build():
    mesh = plsc.VectorSubcoreMesh(core_axis_name="c", subcore_axis_name="s")

    @functools.partial(
        pl.kernel,
        mesh=mesh,
        out_type=[
            jax.ShapeDtypeStruct((B, DB, DIN), jnp.float32),
            jax.ShapeDtypeStruct((B, DB, DIN), jnp.float32),
        ],
        scratch_types=[
            pltpu.VMEM((NODES, DIN), jnp.float32),
            pltpu.VMEM((NODES, DIN), jnp.float32),
            pltpu.VMEM((NODES, DIN), jnp.float32),
            pltpu.VMEM((PAD, DIN), jnp.float32),
            pltpu.VMEM((PAD, DIN), jnp.float32),
            pltpu.SemaphoreType.DMA,
            pltpu.SemaphoreType.DMA,
            pltpu.SemaphoreType.DMA,
            pltpu.SemaphoreType.DMA,
            pltpu.SemaphoreType.DMA,
            pltpu.SemaphoreType.DMA,
        ],
    )
    def sc_prep(uf_hbm, if_hbm, du_hbm, di_hbm,
                bufa, bufb, bufc, ob0, ob1, sia, sib, sic, so0, so1, sh):
        wid = lax.axis_index("s") * 2 + lax.axis_index("c")
        base = wid * IPW

        def compute(buf, ob):
            def m0(k, acc):
                return tuple(acc[v] + buf[k, pl.ds(16 * v, 16)]
                             for v in range(VPR))
            acc = lax.fori_loop(
                2, 1 + N1, m0,
                tuple(buf[1, pl.ds(16 * v, 16)] for v in range(VPR)))
            for v in range(VPR):
                ob[0, pl.ds(16 * v, 16)] = acc[v] * (1.0 / N1)

            def seg(j, c):
                r0 = 1 + N1 + N2 * j
                for v in range(VPR):
                    a = buf[r0, pl.ds(16 * v, 16)]
                    for k in range(1, N2):
                        a = a + buf[r0 + k, pl.ds(16 * v, 16)]
                    ob[1 + j, pl.ds(16 * v, 16)] = a * (1.0 / N2)
                return c
            lax.fori_loop(0, N1, seg, 0)

        def unit(buf, sem_i, ob, sem_o, feat, out, b, nxt):
            pltpu.make_async_copy(feat.at[b], buf, sem_i).wait()

            @pl.when(b - base >= 2)
            def _():
                pltpu.make_async_copy(ob, out.at[b - 2, pl.ds(PAD, PAD), :],
                                      sem_o).wait()
            compute(buf, ob)
            pltpu.make_async_copy(buf.at[pl.ds(0, PAD)],
                                  out.at[b, pl.ds(0, PAD), :], sh).start()
            pltpu.make_async_copy(ob, out.at[b, pl.ds(PAD, PAD), :],
                                  sem_o).start()
            pltpu.make_async_copy(buf.at[pl.ds(0, PAD)],
                                  out.at[b, pl.ds(0, PAD), :], sh).wait()

            @pl.when(nxt < base + IPW)
            def _():
                pltpu.make_async_copy(feat.at[nxt], buf, sem_i).start()

        for feat, out in ((uf_hbm, du_hbm), (if_hbm, di_hbm)):
            pltpu.make_async_copy(feat.at[base], bufa, sia).start()
            pltpu.make_async_copy(feat.at[base + 1], bufb, sib).start()
            pltpu.make_async_copy(feat.at[base + 2], bufc, sic).start()

            def body(g, carry, feat=feat, out=out):
                p = base + 3 * g
                unit(bufa, sia, ob0, so0, feat, out, p, p + 3)
                unit(bufb, sib, ob1, so1, feat, out, p + 1, p + 4)
                unit(bufc, sic, ob0, so0, feat, out, p + 2, p + 5)
                return carry

            lax.fori_loop(0, IPW // 3, body, 0)
            p = base + IPW - 2
            unit(bufa, sia, ob0, so0, feat, out, p, base + IPW)
            unit(bufb, sib, ob1, so1, feat, out, p + 1, base + IPW)
            pltpu.make_async_copy(ob0, out.at[p, pl.ds(PAD, PAD), :],
                                  so0).wait()
            pltpu.make_async_copy(ob1, out.at[p + 1, pl.ds(PAD, PAD), :],
                                  so1).wait()

    return sc_prep


_sc_prep = _sc_prep_build()


def _leaky(x):
    return jnp.where(x >= 0, x, x * 0.01)


def _gnn_tail(h32, n32, w1a, w1b, b1, w2a, w2b, b2):
    hf = h32.reshape(BB * PAD, DIN)
    nf = n32.reshape(BB * PAD, DIN)
    l1 = _leaky(
        jnp.dot(hf, w1a, preferred_element_type=jnp.float32)
        + jnp.dot(nf, w1b, preferred_element_type=jnp.float32)
        + b1
    ).reshape(BB, PAD, H0)

    h0n = l1[:, 0, :]
    neigh = jnp.mean(l1[:, 1:1 + N1, :], axis=1)
    h0f = _leaky(
        jnp.dot(h0n, w2a, preferred_element_type=jnp.float32)
        + jnp.dot(neigh, w2b, preferred_element_type=jnp.float32)
        + b2
    )
    return _leaky(h0f)


def _head(uh, ih, wl, bl, out_ref):
    p = uh * ih
    out_ref[...] = jax.nn.sigmoid(
        jnp.dot(p, wl, preferred_element_type=jnp.float32) + bl)


def _dense_kernel(du_ref, di_ref, w1ua_ref, w1ub_ref, b1u_ref, w2ua_ref,
                  w2ub_ref, b2u_ref, w1ia_ref, w1ib_ref, b1i_ref, w2ia_ref,
                  w2ib_ref, b2i_ref, wl_ref, bl_ref, out_ref):
    du, di = du_ref[...], di_ref[...]
    uh = _gnn_tail(du[:, 0:PAD, :], du[:, PAD:DB, :],
                   w1ua_ref[...], w1ub_ref[...], b1u_ref[...],
                   w2ua_ref[...], w2ub_ref[...], b2u_ref[...])
    ih = _gnn_tail(di[:, 0:PAD, :], di[:, PAD:DB, :],
                   w1ia_ref[...], w1ib_ref[...], b1i_ref[...],
                   w2ia_ref[...], w2ib_ref[...], b2i_ref[...])
    _head(uh, ih, wl_ref[...], bl_ref[...], out_ref)


def kernel(sampling_user_feat, sampling_item_feat, W1_u, b1_u, W2_u, b2_u,
           W1_i, b1_i, W2_i, b2_i, W_lin, b_lin):
    dense_u, dense_i = _sc_prep(sampling_user_feat, sampling_item_feat)

    w1ua, w1ub = W1_u[:DIN], W1_u[DIN:]
    w2ua, w2ub = W2_u[:H0], W2_u[H0:]
    w1ia, w1ib = W1_i[:DIN], W1_i[DIN:]
    w2ia, w2ib = W2_i[:H0], W2_i[H0:]
    b1u = b1_u.reshape(1, H0)
    b2u = b2_u.reshape(1, H1)
    b1i = b1_i.reshape(1, H0)
    b2i = b2_i.reshape(1, H1)
    wl = jnp.zeros((H1, 128), jnp.float32).at[:, :2].set(W_lin)
    bl = jnp.zeros((1, 128), jnp.float32).at[:, :2].set(b_lin)

    def wspec(shape):
        return pl.BlockSpec(shape, lambda i: tuple(0 for _ in shape))

    wspecs = [
        wspec((DIN, H0)), wspec((DIN, H0)), wspec((1, H0)),
        wspec((H0, H1)), wspec((H0, H1)), wspec((1, H1)),
        wspec((DIN, H0)), wspec((DIN, H0)), wspec((1, H0)),
        wspec((H0, H1)), wspec((H0, H1)), wspec((1, H1)),
        wspec((H1, 128)), wspec((1, 128)),
    ]

    dense_spec = pl.BlockSpec((BB, DB, DIN), lambda i: (i, 0, 0))
    out = pl.pallas_call(
        _dense_kernel,
        grid=(B // BB,),
        in_specs=[dense_spec, dense_spec] + wspecs,
        out_specs=pl.BlockSpec((BB, 128), lambda i: (i, 0)),
        out_shape=jax.ShapeDtypeStruct((B, 128), jnp.float32),
    )(dense_u, dense_i,
      w1ua, w1ub, b1u, w2ua, w2ub, b2u,
      w1ia, w1ib, b1i, w2ia, w2ib, b2i, wl, bl)
    return out[:, :2]


# speedup vs baseline: 1.4712x; 1.4712x over previous
import jax
import jax.numpy as jnp
from jax.experimental import pallas as pl

B = 1024
N1, N2 = 25, 10
DIN = 128
H0, H1 = 256, 128
NODES = 1 + N1 + N1 * N2
BB = 64
PAD = 32


def _leaky(x):
    return jnp.where(x >= 0, x, x * 0.01)


def _tower(f, w1a, w1b, b1, w2a, w2b, b2):
    h32 = f[:, 0:PAD, :]
    parts = [jnp.mean(f[:, 1:1 + N1, :], axis=1, keepdims=True)]
    for j in range(N1):
        lo = 1 + N1 + N2 * j
        parts.append(jnp.mean(f[:, lo:lo + N2, :], axis=1, keepdims=True))
    parts.append(jnp.zeros((BB, PAD - 1 - N1, DIN), jnp.float32))
    n32 = jnp.concatenate(parts, axis=1)

    hf = h32.reshape(BB * PAD, DIN)
    nf = n32.reshape(BB * PAD, DIN)
    l1 = _leaky(
        jnp.dot(hf, w1a, preferred_element_type=jnp.float32)
        + jnp.dot(nf, w1b, preferred_element_type=jnp.float32)
        + b1
    ).reshape(BB, PAD, H0)

    h0n = l1[:, 0, :]
    neigh = jnp.mean(l1[:, 1:1 + N1, :], axis=1)
    h0f = _leaky(
        jnp.dot(h0n, w2a, preferred_element_type=jnp.float32)
        + jnp.dot(neigh, w2b, preferred_element_type=jnp.float32)
        + b2
    )
    return _leaky(h0f)


def _fused_kernel(uf_ref, if_ref, w1ua_ref, w1ub_ref, b1u_ref, w2ua_ref,
                  w2ub_ref, b2u_ref, w1ia_ref, w1ib_ref, b1i_ref, w2ia_ref,
                  w2ib_ref, b2i_ref, wl_ref, bl_ref, out_ref):
    uh = _tower(uf_ref[...], w1ua_ref[...], w1ub_ref[...], b1u_ref[...],
                w2ua_ref[...], w2ub_ref[...], b2u_ref[...])
    ih = _tower(if_ref[...], w1ia_ref[...], w1ib_ref[...], b1i_ref[...],
                w2ia_ref[...], w2ib_ref[...], b2i_ref[...])
    p = uh * ih
    z = jnp.dot(p, wl_ref[...], preferred_element_type=jnp.float32) + bl_ref[...]
    out_ref[...] = jax.nn.sigmoid(z)


def kernel(sampling_user_feat, sampling_item_feat, W1_u, b1_u, W2_u, b2_u,
           W1_i, b1_i, W2_i, b2_i, W_lin, b_lin):
    w1ua, w1ub = W1_u[:DIN], W1_u[DIN:]
    w2ua, w2ub = W2_u[:H0], W2_u[H0:]
    w1ia, w1ib = W1_i[:DIN], W1_i[DIN:]
    w2ia, w2ib = W2_i[:H0], W2_i[H0:]
    b1u = b1_u.reshape(1, H0)
    b2u = b2_u.reshape(1, H1)
    b1i = b1_i.reshape(1, H0)
    b2i = b2_i.reshape(1, H1)
    wl = jnp.zeros((H1, 128), jnp.float32).at[:, :2].set(W_lin)
    bl = jnp.zeros((1, 128), jnp.float32).at[:, :2].set(b_lin)

    grid = B // BB
    feat_spec = pl.BlockSpec((BB, NODES, DIN), lambda i: (i, 0, 0))

    def wspec(shape):
        return pl.BlockSpec(shape, lambda i: tuple(0 for _ in shape))

    out = pl.pallas_call(
        _fused_kernel,
        grid=(grid,),
        in_specs=[
            feat_spec, feat_spec,
            wspec((DIN, H0)), wspec((DIN, H0)), wspec((1, H0)),
            wspec((H0, H1)), wspec((H0, H1)), wspec((1, H1)),
            wspec((DIN, H0)), wspec((DIN, H0)), wspec((1, H0)),
            wspec((H0, H1)), wspec((H0, H1)), wspec((1, H1)),
            wspec((H1, 128)), wspec((1, 128)),
        ],
        out_specs=pl.BlockSpec((BB, 128), lambda i: (i, 0)),
        out_shape=jax.ShapeDtypeStruct((B, 128), jnp.float32),
    )(sampling_user_feat, sampling_item_feat,
      w1ua, w1ub, b1u, w2ua, w2ub, b2u,
      w1ia, w1ib, b1i, w2ia, w2ib, b2i, wl, bl)
    return out[:, :2]


# speedup vs baseline: 1.5166x; 1.0309x over previous
import jax
import jax.numpy as jnp
from jax.experimental import pallas as pl

B = 1024
N1, N2 = 25, 10
DIN = 128
H0, H1 = 256, 128
NODES = 1 + N1 + N1 * N2
BB = 64
PAD = 32


def _leaky(x):
    return jnp.where(x >= 0, x, x * 0.01)


def _tower(f, w1, b1, w2, b2):
    h32 = f[:, 0:PAD, :]
    parts = [jnp.mean(f[:, 1:1 + N1, :], axis=1, keepdims=True)]
    for j in range(N1):
        lo = 1 + N1 + N2 * j
        parts.append(jnp.mean(f[:, lo:lo + N2, :], axis=1, keepdims=True))
    parts.append(jnp.zeros((BB, PAD - 1 - N1, DIN), jnp.float32))
    n32 = jnp.concatenate(parts, axis=1)

    x = jnp.concatenate([h32, n32], axis=-1)
    l1 = _leaky(
        jnp.dot(x.reshape(BB * PAD, 2 * DIN), w1,
                preferred_element_type=jnp.float32)
        + b1
    ).reshape(BB, PAD, H0)

    h0n = l1[:, 0, :]
    neigh = jnp.mean(l1[:, 1:1 + N1, :], axis=1)
    h0f = _leaky(
        jnp.dot(jnp.concatenate([h0n, neigh], axis=-1), w2,
                preferred_element_type=jnp.float32)
        + b2
    )
    return _leaky(h0f)


def _fused_kernel(uf_ref, if_ref, w1u_ref, b1u_ref, w2u_ref, b2u_ref,
                  w1i_ref, b1i_ref, w2i_ref, b2i_ref,
                  wl_ref, bl_ref, out_ref):
    uh = _tower(uf_ref[...], w1u_ref[...], b1u_ref[...],
                w2u_ref[...], b2u_ref[...])
    ih = _tower(if_ref[...], w1i_ref[...], b1i_ref[...],
                w2i_ref[...], b2i_ref[...])
    p = uh * ih
    z = jnp.dot(p, wl_ref[...], preferred_element_type=jnp.float32) + bl_ref[...]
    out_ref[...] = jax.nn.sigmoid(z)


def kernel(sampling_user_feat, sampling_item_feat, W1_u, b1_u, W2_u, b2_u,
           W1_i, b1_i, W2_i, b2_i, W_lin, b_lin):
    b1u = b1_u.reshape(1, H0)
    b2u = b2_u.reshape(1, H1)
    b1i = b1_i.reshape(1, H0)
    b2i = b2_i.reshape(1, H1)
    wl = jnp.zeros((H1, 128), jnp.float32).at[:, :2].set(W_lin)
    bl = jnp.zeros((1, 128), jnp.float32).at[:, :2].set(b_lin)

    grid = B // BB
    feat_spec = pl.BlockSpec((BB, NODES, DIN), lambda i: (i, 0, 0))

    def wspec(shape):
        return pl.BlockSpec(shape, lambda i: tuple(0 for _ in shape))

    out = pl.pallas_call(
        _fused_kernel,
        grid=(grid,),
        in_specs=[
            feat_spec, feat_spec,
            wspec((2 * DIN, H0)), wspec((1, H0)),
            wspec((2 * H0, H1)), wspec((1, H1)),
            wspec((2 * DIN, H0)), wspec((1, H0)),
            wspec((2 * H0, H1)), wspec((1, H1)),
            wspec((H1, 128)), wspec((1, 128)),
        ],
        out_specs=pl.BlockSpec((BB, 128), lambda i: (i, 0)),
        out_shape=jax.ShapeDtypeStruct((B, 128), jnp.float32),
    )(sampling_user_feat, sampling_item_feat,
      W1_u, b1u, W2_u, b2u, W1_i, b1i, W2_i, b2i, wl, bl)
    return out[:, :2]
